# baseline (speedup 1.0000x reference)
import functools

import jax
import jax.numpy as jnp
from jax import lax
from jax.experimental import pallas as pl
from jax.experimental.pallas import tpu as pltpu
from jax.experimental.pallas import tpu_sc as plsc

N = 10000
E = 160000
G = 128
NT = 512
DH = 256

NC = 2
NS = 16
NW = NC * NS

EP = 163840

_mesh = functools.partial(
    plsc.VectorSubcoreMesh,
    core_axis_name="c",
    subcore_axis_name="s",
    num_cores=NC,
    num_subcores=NS,
)

SUBQ = 632


def _splat16(val):
    return jnp.zeros((16,), jnp.int32) + val


def _lane(vec16, j):
    return lax.gather(
        vec16,
        _splat16(j)[:, None],
        lax.GatherDimensionNumbers(
            offset_dims=(), collapsed_slice_dims=(0,), start_index_map=(0,)),
        slice_sizes=(1,),
        mode=lax.GatherScatterMode.PROMISE_IN_BOUNDS,
    )


def _emb_gather(embp, xidx, y, b1r):
    QUOTA = 320
    CH = 80

    @functools.partial(
        pl.kernel,
        out_type=jax.ShapeDtypeStruct((N, 128), jnp.float32),
        mesh=_mesh(),
        scratch_types=[
            pltpu.VMEM((CH,), jnp.int32),
            pltpu.VMEM((CH,), jnp.float32),
            pltpu.VMEM((CH, 128), jnp.float32),
            pltpu.VMEM((1, 128), jnp.float32),
            pltpu.SemaphoreType.DMA,
        ],
    )
    def k(emb_h, idx_h, y_h, b_h, out_h, idxv, yv, rows, bvm, sem):
        cid = lax.axis_index("c")
        sid = lax.axis_index("s")
        wid = sid * NC + cid
        base = jnp.minimum(wid * QUOTA, N - QUOTA)
        pltpu.sync_copy(b_h, bvm)
        bb = [bvm[0, pl.ds(16 * c2, 16)] for c2 in range(8)]
        lastlane = lax.iota(jnp.int32, 16) == 15
        for kk in range(QUOTA // CH):
            b2 = base + kk * CH
            pltpu.sync_copy(idx_h.at[pl.ds(b2, CH)], idxv)
            pltpu.sync_copy(y_h.at[pl.ds(b2, CH)], yv)
            pltpu.async_copy(emb_h.at[idxv], rows, sem).wait()

            def grp(g, carry):
                y16 = yv[pl.ds(g * 16, 16)]
                for j in range(16):
                    r = g * 16 + j
                    yj = _lane(y16, j)
                    for c2 in range(8):
                        sl = pl.ds(16 * c2, 16)
                        v = rows[r, sl]
                        if c2 == 7:
                            v = jnp.where(lastlane, yj, v)
                        rows[r, sl] = v + bb[c2]
                return carry

            lax.fori_loop(0, CH // 16, grp, 0)
            pltpu.sync_copy(rows, out_h.at[pl.ds(b2, CH)])

    return k(embp, xidx, y, b1r)


CHE = 64
SECK = 40


def _edge_section(feat_h, aggr, srcv, eav, dstv, bufs, gsems, ssems, wv):
    def prep(k, b):
        pltpu.async_copy(
            feat_h.at[srcv.at[pl.ds(CHE * k, CHE)]], bufs[b], gsems[b])

    def waitg(b):
        pltpu.make_async_copy(
            feat_h.at[srcv.at[pl.ds(0, CHE)]], bufs[b], gsems[b]).wait()

    def scat(k, b):
        pltpu.async_copy(bufs[b], aggr.at[dstv.at[k]], ssems[b], add=True)

    def waits(b):
        pltpu.make_async_copy(bufs[b], aggr.at[dstv.at[0]], ssems[b]).wait()

    def compute(k, b):
        rg = bufs[b]

        def grp(g, carry):
            a16 = eav[pl.ds(CHE * k + g * 16, 16)]
            for j in range(16):
                e = g * 16 + j
                aj = _lane(a16, j)
                for c2 in range(8):
                    sl = pl.ds(16 * c2, 16)
                    rg[e, sl] = jnp.maximum(rg[e, sl] + aj * wv[c2], 0.0)
            return carry

        lax.fori_loop(0, CHE // 16, grp, 0)

    prep(0, 0)
    prep(1, 1)

    def lp(kk, carry):
        for b in range(4):
            k = 4 * kk + b
            bp = (b + 2) % 4

            @pl.when(k >= 2)
            def _():
                waits(bp)

            @pl.when(k + 2 < SECK)
            def _():
                prep(k + 2, bp)

            waitg(b)
            compute(k, b)
            scat(k, b)
        return carry

    lax.fori_loop(0, SECK // 4, lp, 0)
    waits(2)
    waits(3)


def _edge_scratch():
    return (
        [
            pltpu.VMEM_SHARED((N + 128, 128), jnp.float32),
            pltpu.VMEM((SECK * CHE,), jnp.int32),
            pltpu.VMEM((SECK * CHE,), jnp.float32),
            pltpu.VMEM((SECK, CHE), jnp.int32),
        ]
        + [pltpu.VMEM((CHE, 128), jnp.float32) for _ in range(4)]
        + [pltpu.VMEM((1, 128), jnp.float32)]
        + [pltpu.SemaphoreType.DMA] * 8
    )


def _edge_pass1(xy, src, dst2, ea, w1r, zeros):
    @functools.partial(
        pl.kernel,
        out_type=jax.ShapeDtypeStruct((2, N, 128), jnp.float32),
        mesh=_mesh(),
        scratch_types=_edge_scratch(),
    )
    def k(xy_h, src_h, dst_h, ea_h, w_h, z_h, out_h,
          aggr, srcv, eav, dstv, b0, b1, b2, b3, wvm,
          g0, g1, g2, g3, s0, s1, s2, s3):
        cid = lax.axis_index("c")
        sid = lax.axis_index("s")
        wid = sid * NC + cid
        r0 = jnp.minimum(sid * SUBQ, N - SUBQ)
        pltpu.sync_copy(z_h.at[pl.ds(r0, SUBQ)], aggr.at[pl.ds(r0, SUBQ)])
        pltpu.sync_copy(w_h, wvm)
        plsc.subcore_barrier()
        wv = [wvm[0, pl.ds(16 * c2, 16)] for c2 in range(8)]

        def section(s, carry):
            eb = wid * (2 * SECK * CHE) + s * (SECK * CHE)
            pltpu.sync_copy(src_h.at[pl.ds(eb, SECK * CHE)], srcv)
            pltpu.sync_copy(ea_h.at[pl.ds(eb, SECK * CHE)], eav)
            pltpu.sync_copy(
                dst_h.at[pl.ds(wid * (2 * SECK) + s * SECK, SECK)], dstv)
            _edge_section(xy_h, aggr, srcv, eav, dstv,
                          (b0, b1, b2, b3), (g0, g1, g2, g3),
                          (s0, s1, s2, s3), wv)
            return carry

        lax.fori_loop(0, 2, section, 0)
        plsc.subcore_barrier()
        pltpu.sync_copy(aggr.at[pl.ds(r0, SUBQ)],
                        out_h.at[cid, pl.ds(r0, SUBQ)])

    return k(xy, src, dst2, ea, w1r, zeros)


def _edge_pass2(h1a, h1b, src, dst2, ea, w2h, zeros):
    @functools.partial(
        pl.kernel,
        out_type=jax.ShapeDtypeStruct((2, N, 128), jnp.float32),
        mesh=_mesh(),
        scratch_types=_edge_scratch(),
    )
    def k(ha_h, hb_h, src_h, dst_h, ea_h, w_h, z_h, out_h,
          aggr, srcv, eav, dstv, b0, b1, b2, b3, wvm,
          g0, g1, g2, g3, s0, s1, s2, s3):
        cid = lax.axis_index("c")
        sid = lax.axis_index("s")
        r0 = jnp.minimum(sid * SUBQ, N - SUBQ)
        pltpu.sync_copy(z_h.at[pl.ds(r0, SUBQ)], aggr.at[pl.ds(r0, SUBQ)])
        pltpu.sync_copy(w_h.at[pl.ds(cid, 1)], wvm)
        plsc.subcore_barrier()
        wv = [wvm[0, pl.ds(16 * c2, 16)] for c2 in range(8)]

        def section(s, carry):
            eb = sid * (4 * SECK * CHE) + s * (SECK * CHE)
            pltpu.sync_copy(src_h.at[pl.ds(eb, SECK * CHE)], srcv)
            pltpu.sync_copy(ea_h.at[pl.ds(eb, SECK * CHE)], eav)
            pltpu.sync_copy(
                dst_h.at[pl.ds(sid * (4 * SECK) + s * SECK, SECK)], dstv)

            @pl.when(cid == 0)
            def _():
                _edge_section(ha_h, aggr, srcv, eav, dstv,
                              (b0, b1, b2, b3), (g0, g1, g2, g3),
                              (s0, s1, s2, s3), wv)

            @pl.when(cid == 1)
            def _():
                _edge_section(hb_h, aggr, srcv, eav, dstv,
                              (b0, b1, b2, b3), (g0, g1, g2, g3),
                              (s0, s1, s2, s3), wv)

            return carry

        lax.fori_loop(0, 4, section, 0)
        plsc.subcore_barrier()
        pltpu.sync_copy(aggr.at[pl.ds(r0, SUBQ)],
                        out_h.at[cid, pl.ds(r0, SUBQ)])

    return k(h1a, h1b, src, dst2, ea, w2h, zeros)


R = 1000
NB = N // R


def _mlp_a(parts, agg, WT, b):
    DI = WT.shape[0]

    def body(*refs):
        nparts = len(parts)
        part_refs = refs[:nparts]
        agg_r, w_r, b_r, u_r, st_r, acc_r = refs[nparts:]
        i = pl.program_id(0)
        if nparts == 1:
            z = part_refs[0][...]
        else:
            z = jnp.concatenate([p[...] for p in part_refs], axis=1)
        z = z + jnp.concatenate([agg_r[0], agg_r[1]], axis=1) \
            if DI == 256 else z + agg_r[0] + agg_r[1]
        u = jnp.dot(z, w_r[...], preferred_element_type=jnp.float32) + b_r[...]
        u_r[...] = u
        s1 = jnp.sum(u, axis=0, keepdims=True)
        s2 = jnp.sum(u * u, axis=0, keepdims=True)
        st = jnp.concatenate([s1, s2], axis=0)

        @pl.when(i == 0)
        def _():
            acc_r[...] = st

        @pl.when(i > 0)
        def _():
            acc_r[...] = acc_r[...] + st

        @pl.when(i == NB - 1)
        def _():
            st_r[...] = acc_r[...]

    in_specs = (
        [pl.BlockSpec((R, p.shape[1]), lambda i: (i, 0)) for p in parts]
        + [
            pl.BlockSpec((2, R, 128), lambda i: (0, i, 0)),
            pl.BlockSpec((DI, DH), lambda i: (0, 0)),
            pl.BlockSpec((1, DH), lambda i: (0, 0)),
        ]
    )
    return pl.pallas_call(
        body,
        grid=(NB,),
        in_specs=in_specs,
        out_specs=[
            pl.BlockSpec((R, DH), lambda i: (i, 0)),
            pl.BlockSpec((2, DH), lambda i: (0, 0)),
        ],
        out_shape=[
            jax.ShapeDtypeStruct((N, DH), jnp.float32),
            jax.ShapeDtypeStruct((2, DH), jnp.float32),
        ],
        scratch_shapes=[pltpu.VMEM((2, DH), jnp.float32)],
    )(*parts, agg, WT, b)


def _mlp_b(u, stats, g, bt, WT, b2, badd):
    def body(u_r, st_r, g_r, bt_r, w_r, b_r, ba_r, ha_r, hb_r):
        st = st_r[...]
        m = st[0:1, :] / N
        v = st[1:2, :] / N - m * m
        inv = lax.rsqrt(v + 1e-5)
        t = jnp.maximum((u_r[...] - m) * inv * g_r[...] + bt_r[...], 0.0)
        h = jnp.dot(t, w_r[...], preferred_element_type=jnp.float32) + b_r[...]
        h = jnp.maximum(h, 0.0) + ba_r[...]
        ha_r[...] = h[:, :128]
        hb_r[...] = h[:, 128:]

    full = lambda shape: pl.BlockSpec(shape, lambda i: tuple(0 for _ in shape))
    return pl.pallas_call(
        body,
        grid=(NB,),
        in_specs=[
            pl.BlockSpec((R, DH), lambda i: (i, 0)),
            full((2, DH)), full((1, DH)), full((1, DH)),
            full((DH, DH)), full((1, DH)), full((1, DH)),
        ],
        out_specs=[
            pl.BlockSpec((R, 128), lambda i: (i, 0)),
            pl.BlockSpec((R, 128), lambda i: (i, 0)),
        ],
        out_shape=[
            jax.ShapeDtypeStruct((N, 128), jnp.float32),
            jax.ShapeDtypeStruct((N, 128), jnp.float32),
        ],
    )(u, stats, g, bt, WT, b2, badd)


def _pool_heads(batch3, h1a, h1b, h2a, h2b, bsub,
                Wf1T, bf1, Wf2T, bf2, Wb1T, bb1, Wb2T, bb2):
    def body(b_r, h1a_r, h1b_r, h2a_r, h2b_r, bs_r,
             wf1_r, bf1_r, wf2_r, bf2_r, wb1_r, bb1_r, wb2_r, bb2_r,
             lf_r, lb_r, pacc):
        i = pl.program_id(0)

        @pl.when(i == 0)
        def _():
            pacc[...] = jnp.zeros((G, 2 * DH), jnp.float32)

        bb = b_r[0]
        oh = (lax.broadcasted_iota(jnp.int32, (G, R), 0) == bb)
        oh = oh.astype(jnp.float32)
        hs = (h1a_r[...] - bs_r[:, :128], h1b_r[...] - bs_r[:, 128:],
              h2a_r[...], h2b_r[...])
        for idx, h in enumerate(hs):
            sl = pl.ds(128 * idx, 128)
            pacc[:, sl] = pacc[:, sl] + jnp.dot(
                oh, h, preferred_element_type=jnp.float32)

        @pl.when(i == NB - 1)
        def _():
            h = pacc[...]
            tf = jnp.maximum(
                jnp.dot(h, wf1_r[...], preferred_element_type=jnp.float32)
                + bf1_r[...], 0.0)
            lf_r[...] = jnp.dot(
                tf, wf2_r[...], preferred_element_type=jnp.float32) + bf2_r[...]
            tb = jnp.maximum(
                jnp.dot(h, wb1_r[...], preferred_element_type=jnp.float32)
                + bb1_r[...], 0.0)
            lb_r[...] = jnp.dot(
                tb, wb2_r[...], preferred_element_type=jnp.float32) + bb2_r[...]

    full = lambda shape: pl.BlockSpec(shape, lambda i: tuple(0 for _ in shape))
    return pl.pallas_call(
        body,
        grid=(NB,),
        in_specs=[
            pl.BlockSpec((1, 1, R), lambda i: (i, 0, 0)),
            pl.BlockSpec((R, 128), lambda i: (i, 0)),
            pl.BlockSpec((R, 128), lambda i: (i, 0)),
            pl.BlockSpec((R, 128), lambda i: (i, 0)),
            pl.BlockSpec((R, 128), lambda i: (i, 0)),
            full((1, DH)),
            full((2 * DH, DH)), full((1, DH)),
            full((DH, NT)), full((1, NT)),
            full((2 * DH, DH)), full((1, DH)),
            full((DH, NT)), full((1, NT)),
        ],
        out_specs=[full((G, NT)), full((G, NT))],
        out_shape=[
            jax.ShapeDtypeStruct((G, NT), jnp.float32),
            jax.ShapeDtypeStruct((G, NT), jnp.float32),
        ],
        scratch_shapes=[pltpu.VMEM((G, 2 * DH), jnp.float32)],
    )(batch3, h1a, h1b, h2a, h2b, bsub,
      Wf1T, bf1, Wf2T, bf2, Wb1T, bb1, Wb2T, bb2)


def kernel(x, y, edge_index, edge_attr, batch, emb, We1, be1, W1a, b1a, g1,
           bt1, W1b, b1b, We2, be2, W2a, b2a, g2, bt2, W2b, b2b, Wf1, bf1,
           Wf2, bf2, Wb1, bb1, Wb2, bb2):
    f32 = jnp.float32
    embp = jnp.pad(emb.astype(f32), ((0, 0), (0, 1)))
    xidx = x.reshape(-1).astype(jnp.int32)
    pad = EP - E
    src = jnp.pad(edge_index[0].astype(jnp.int32), (0, pad))
    dst = jnp.concatenate([
        edge_index[1].astype(jnp.int32),
        N + (jnp.arange(pad, dtype=jnp.int32) % 128),
    ])
    dst2 = dst.reshape(EP // CHE, CHE)
    ea = jnp.pad(edge_attr.reshape(-1).astype(f32), (0, pad))
    zeros = jnp.zeros((N, 128), f32)

    w1r = We1[:, 0].reshape(1, 128)
    b1r = be1.reshape(1, 128)
    w2h = We2[:, 0].reshape(2, 128)
    be2r = be2.reshape(1, -1)
    b1a_c = (b1a - be1 @ W1a.T).reshape(1, -1)
    b2a_c = (b2a - be2 @ W2a.T).reshape(1, -1)

    xy = _emb_gather(embp, xidx, y, b1r)
    pagg1 = _edge_pass1(xy, src, dst2, ea, w1r, zeros)

    u1, st1 = _mlp_a([xy], pagg1, W1a.T, b1a_c)
    h1a, h1b = _mlp_b(u1, st1, g1.reshape(1, -1), bt1.reshape(1, -1),
                      W1b.T, b1b.reshape(1, -1), be2r)

    agg2 = _edge_pass2(h1a, h1b, src, dst2, ea, w2h, zeros)
    u2, st2 = _mlp_a([h1a, h1b], agg2, W2a.T, b2a_c)
    h2a, h2b = _mlp_b(u2, st2, g2.reshape(1, -1), bt2.reshape(1, -1),
                      W2b.T, b2b.reshape(1, -1), jnp.zeros((1, DH), f32))

    batch3 = batch.reshape(NB, 1, R).astype(jnp.int32)
    lf, lb = _pool_heads(batch3, h1a, h1b, h2a, h2b, be2r,
                         Wf1.T, bf1.reshape(1, -1), Wf2.T, bf2.reshape(1, -1),
                         Wb1.T, bb1.reshape(1, -1), Wb2.T, bb2.reshape(1, -1))
    return (lf, lb)


# speedup vs baseline: 1.9167x; 1.9167x over previous
import functools

import jax
import jax.numpy as jnp
from jax import lax
from jax.experimental import pallas as pl
from jax.experimental.pallas import tpu as pltpu
from jax.experimental.pallas import tpu_sc as plsc

N = 10000
E = 160000
G = 128
NT = 512
DH = 256

NC = 2
NS = 16
NW = NC * NS

EP = 163840

_mesh = functools.partial(
    plsc.VectorSubcoreMesh,
    core_axis_name="c",
    subcore_axis_name="s",
    num_cores=NC,
    num_subcores=NS,
)

SUBQ = 632


def _splat16(val):
    return jnp.zeros((16,), jnp.int32) + val


def _lane(vec16, j):
    return lax.gather(
        vec16,
        _splat16(j)[:, None],
        lax.GatherDimensionNumbers(
            offset_dims=(), collapsed_slice_dims=(0,), start_index_map=(0,)),
        slice_sizes=(1,),
        mode=lax.GatherScatterMode.PROMISE_IN_BOUNDS,
    )


def _emb_gather(embp, xidx, y, b1r):
    QUOTA = 320
    CH = 80

    @functools.partial(
        pl.kernel,
        out_type=jax.ShapeDtypeStruct((N, 128), jnp.float32),
        mesh=_mesh(),
        scratch_types=[
            pltpu.VMEM((CH,), jnp.int32),
            pltpu.VMEM((CH,), jnp.float32),
            pltpu.VMEM((CH, 128), jnp.float32),
            pltpu.VMEM((1, 128), jnp.float32),
            pltpu.SemaphoreType.DMA,
        ],
    )
    def k(emb_h, idx_h, y_h, b_h, out_h, idxv, yv, rows, bvm, sem):
        cid = lax.axis_index("c")
        sid = lax.axis_index("s")
        wid = sid * NC + cid
        base = jnp.minimum(wid * QUOTA, N - QUOTA)
        pltpu.sync_copy(b_h, bvm)
        bb = [bvm[0, pl.ds(16 * c2, 16)] for c2 in range(8)]
        lastlane = lax.iota(jnp.int32, 16) == 15
        for kk in range(QUOTA // CH):
            b2 = base + kk * CH
            pltpu.sync_copy(idx_h.at[pl.ds(b2, CH)], idxv)
            pltpu.sync_copy(y_h.at[pl.ds(b2, CH)], yv)
            pltpu.async_copy(emb_h.at[idxv], rows, sem).wait()

            def grp(g, carry):
                y16 = yv[pl.ds(g * 16, 16)]
                for j in range(16):
                    r = g * 16 + j
                    yj = _lane(y16, j)
                    for c2 in range(8):
                        sl = pl.ds(16 * c2, 16)
                        v = rows[r, sl]
                        if c2 == 7:
                            v = jnp.where(lastlane, yj, v)
                        rows[r, sl] = v + bb[c2]
                return carry

            lax.fori_loop(0, CH // 16, grp, 0)
            pltpu.sync_copy(rows, out_h.at[pl.ds(b2, CH)])

    return k(embp, xidx, y, b1r)


CHE = 64
SECK = 40


def _edge_section(feat_h, aggr, srcv, eav, dstv, bufs, gsems, ssems, wv):
    def prep(k, b):
        pltpu.async_copy(
            feat_h.at[srcv.at[pl.ds(CHE * k, CHE)]], bufs[b], gsems[b])

    def waitg(b):
        pltpu.make_async_copy(
            feat_h.at[srcv.at[pl.ds(0, CHE)]], bufs[b], gsems[b]).wait()

    def scat(k, b):
        pltpu.async_copy(bufs[b], aggr.at[dstv.at[k]], ssems[b], add=True)

    def waits(b):
        pltpu.make_async_copy(bufs[b], aggr.at[dstv.at[0]], ssems[b]).wait()

    def compute(k, b):
        rg = bufs[b]

        def grp(g, carry):
            a16 = eav[pl.ds(CHE * k + g * 16, 16)]
            for j in range(16):
                e = g * 16 + j
                aj = _lane(a16, j)
                for c2 in range(8):
                    sl = pl.ds(16 * c2, 16)
                    rg[e, sl] = jnp.maximum(rg[e, sl] + aj * wv[c2], 0.0)
            return carry

        lax.fori_loop(0, CHE // 16, grp, 0)

    prep(0, 0)
    prep(1, 1)

    def lp(kk, carry):
        for b in range(4):
            k = 4 * kk + b
            bp = (b + 2) % 4

            @pl.when(k >= 2)
            def _():
                waits(bp)

            @pl.when(k + 2 < SECK)
            def _():
                prep(k + 2, bp)

            waitg(b)
            compute(k, b)
            scat(k, b)
        return carry

    lax.fori_loop(0, SECK // 4, lp, 0)
    waits(2)
    waits(3)


def _edge_scratch():
    return (
        [
            pltpu.VMEM_SHARED((N + 128, 128), jnp.float32),
            pltpu.VMEM((SECK * CHE,), jnp.int32),
            pltpu.VMEM((SECK * CHE,), jnp.float32),
            pltpu.VMEM((SECK, CHE), jnp.int32),
        ]
        + [pltpu.VMEM((CHE, 128), jnp.float32) for _ in range(4)]
        + [pltpu.VMEM((1, 128), jnp.float32)]
        + [pltpu.SemaphoreType.DMA] * 8
    )


def _edge_pass1(xy, src, dst2, ea, w1r, zeros):
    @functools.partial(
        pl.kernel,
        out_type=jax.ShapeDtypeStruct((2, N, 128), jnp.float32),
        mesh=_mesh(),
        scratch_types=_edge_scratch(),
    )
    def k(xy_h, src_h, dst_h, ea_h, w_h, z_h, out_h,
          aggr, srcv, eav, dstv, b0, b1, b2, b3, wvm,
          g0, g1, g2, g3, s0, s1, s2, s3):
        cid = lax.axis_index("c")
        sid = lax.axis_index("s")
        wid = sid * NC + cid
        r0 = jnp.minimum(sid * SUBQ, N - SUBQ)
        pltpu.sync_copy(z_h.at[pl.ds(r0, SUBQ)], aggr.at[pl.ds(r0, SUBQ)])
        pltpu.sync_copy(w_h, wvm)
        plsc.subcore_barrier()
        wv = [wvm[0, pl.ds(16 * c2, 16)] for c2 in range(8)]

        def section(s, carry):
            eb = wid * (2 * SECK * CHE) + s * (SECK * CHE)
            pltpu.sync_copy(src_h.at[pl.ds(eb, SECK * CHE)], srcv)
            pltpu.sync_copy(ea_h.at[pl.ds(eb, SECK * CHE)], eav)
            pltpu.sync_copy(
                dst_h.at[pl.ds(wid * (2 * SECK) + s * SECK, SECK)], dstv)
            _edge_section(xy_h, aggr, srcv, eav, dstv,
                          (b0, b1, b2, b3), (g0, g1, g2, g3),
                          (s0, s1, s2, s3), wv)
            return carry

        lax.fori_loop(0, 2, section, 0)
        plsc.subcore_barrier()
        pltpu.sync_copy(aggr.at[pl.ds(r0, SUBQ)],
                        out_h.at[cid, pl.ds(r0, SUBQ)])

    return k(xy, src, dst2, ea, w1r, zeros)


def _edge_pass2(h1a, h1b, src, dst2, ea, w2h, zeros):
    @functools.partial(
        pl.kernel,
        out_type=jax.ShapeDtypeStruct((2, N, 128), jnp.float32),
        mesh=_mesh(),
        scratch_types=_edge_scratch(),
    )
    def k(ha_h, hb_h, src_h, dst_h, ea_h, w_h, z_h, out_h,
          aggr, srcv, eav, dstv, b0, b1, b2, b3, wvm,
          g0, g1, g2, g3, s0, s1, s2, s3):
        cid = lax.axis_index("c")
        sid = lax.axis_index("s")
        r0 = jnp.minimum(sid * SUBQ, N - SUBQ)
        pltpu.sync_copy(z_h.at[pl.ds(r0, SUBQ)], aggr.at[pl.ds(r0, SUBQ)])
        pltpu.sync_copy(w_h.at[pl.ds(cid, 1)], wvm)
        plsc.subcore_barrier()
        wv = [wvm[0, pl.ds(16 * c2, 16)] for c2 in range(8)]

        def section(s, carry):
            eb = sid * (4 * SECK * CHE) + s * (SECK * CHE)
            pltpu.sync_copy(src_h.at[pl.ds(eb, SECK * CHE)], srcv)
            pltpu.sync_copy(ea_h.at[pl.ds(eb, SECK * CHE)], eav)
            pltpu.sync_copy(
                dst_h.at[pl.ds(sid * (4 * SECK) + s * SECK, SECK)], dstv)

            @pl.when(cid == 0)
            def _():
                _edge_section(ha_h, aggr, srcv, eav, dstv,
                              (b0, b1, b2, b3), (g0, g1, g2, g3),
                              (s0, s1, s2, s3), wv)

            @pl.when(cid == 1)
            def _():
                _edge_section(hb_h, aggr, srcv, eav, dstv,
                              (b0, b1, b2, b3), (g0, g1, g2, g3),
                              (s0, s1, s2, s3), wv)

            return carry

        lax.fori_loop(0, 4, section, 0)
        plsc.subcore_barrier()
        pltpu.sync_copy(aggr.at[pl.ds(r0, SUBQ)],
                        out_h.at[cid, pl.ds(r0, SUBQ)])

    return k(h1a, h1b, src, dst2, ea, w2h, zeros)


R = 1000
NB = N // R


def _mlp_a(parts, agg, WT, b):
    DI = WT.shape[0]

    def body(*refs):
        nparts = len(parts)
        part_refs = refs[:nparts]
        agg_r, w_r, b_r, u_r, st_r, acc_r = refs[nparts:]
        i = pl.program_id(0)
        if nparts == 1:
            z = part_refs[0][...]
        else:
            z = jnp.concatenate([p[...] for p in part_refs], axis=1)
        z = z + jnp.concatenate([agg_r[0], agg_r[1]], axis=1) \
            if DI == 256 else z + agg_r[0] + agg_r[1]
        u = jnp.dot(z, w_r[...], preferred_element_type=jnp.float32) + b_r[...]
        u_r[...] = u
        s1 = jnp.sum(u, axis=0, keepdims=True)
        s2 = jnp.sum(u * u, axis=0, keepdims=True)
        st = jnp.concatenate([s1, s2], axis=0)

        @pl.when(i == 0)
        def _():
            acc_r[...] = st

        @pl.when(i > 0)
        def _():
            acc_r[...] = acc_r[...] + st

        @pl.when(i == NB - 1)
        def _():
            st_r[...] = acc_r[...]

    in_specs = (
        [pl.BlockSpec((R, p.shape[1]), lambda i: (i, 0)) for p in parts]
        + [
            pl.BlockSpec((2, R, 128), lambda i: (0, i, 0)),
            pl.BlockSpec((DI, DH), lambda i: (0, 0)),
            pl.BlockSpec((1, DH), lambda i: (0, 0)),
        ]
    )
    return pl.pallas_call(
        body,
        grid=(NB,),
        in_specs=in_specs,
        out_specs=[
            pl.BlockSpec((R, DH), lambda i: (i, 0)),
            pl.BlockSpec((2, DH), lambda i: (0, 0)),
        ],
        out_shape=[
            jax.ShapeDtypeStruct((N, DH), jnp.float32),
            jax.ShapeDtypeStruct((2, DH), jnp.float32),
        ],
        scratch_shapes=[pltpu.VMEM((2, DH), jnp.float32)],
    )(*parts, agg, WT, b)


def _mlp_b(u, stats, g, bt, WT, b2, badd):
    def body(u_r, st_r, g_r, bt_r, w_r, b_r, ba_r, ha_r, hb_r):
        st = st_r[...]
        m = st[0:1, :] / N
        v = st[1:2, :] / N - m * m
        inv = lax.rsqrt(v + 1e-5)
        t = jnp.maximum((u_r[...] - m) * inv * g_r[...] + bt_r[...], 0.0)
        h = jnp.dot(t, w_r[...], preferred_element_type=jnp.float32) + b_r[...]
        h = jnp.maximum(h, 0.0) + ba_r[...]
        ha_r[...] = h[:, :128]
        hb_r[...] = h[:, 128:]

    full = lambda shape: pl.BlockSpec(shape, lambda i: tuple(0 for _ in shape))
    return pl.pallas_call(
        body,
        grid=(NB,),
        in_specs=[
            pl.BlockSpec((R, DH), lambda i: (i, 0)),
            full((2, DH)), full((1, DH)), full((1, DH)),
            full((DH, DH)), full((1, DH)), full((1, DH)),
        ],
        out_specs=[
            pl.BlockSpec((R, 128), lambda i: (i, 0)),
            pl.BlockSpec((R, 128), lambda i: (i, 0)),
        ],
        out_shape=[
            jax.ShapeDtypeStruct((N, 128), jnp.float32),
            jax.ShapeDtypeStruct((N, 128), jnp.float32),
        ],
    )(u, stats, g, bt, WT, b2, badd)


def _pool_heads(batch3, h1a, h1b, h2a, h2b, bsub,
                Wf1T, bf1, Wf2T, bf2, Wb1T, bb1, Wb2T, bb2):
    def body(b_r, h1a_r, h1b_r, h2a_r, h2b_r, bs_r,
             wf1_r, bf1_r, wf2_r, bf2_r, wb1_r, bb1_r, wb2_r, bb2_r,
             lf_r, lb_r, pacc):
        i = pl.program_id(0)

        @pl.when(i == 0)
        def _():
            pacc[...] = jnp.zeros((G, 2 * DH), jnp.float32)

        bb = b_r[0]
        oh = (lax.broadcasted_iota(jnp.int32, (G, R), 0) == bb)
        oh = oh.astype(jnp.float32)
        hs = (h1a_r[...] - bs_r[:, :128], h1b_r[...] - bs_r[:, 128:],
              h2a_r[...], h2b_r[...])
        for idx, h in enumerate(hs):
            sl = pl.ds(128 * idx, 128)
            pacc[:, sl] = pacc[:, sl] + jnp.dot(
                oh, h, preferred_element_type=jnp.float32)

        @pl.when(i == NB - 1)
        def _():
            h = pacc[...]
            tf = jnp.maximum(
                jnp.dot(h, wf1_r[...], preferred_element_type=jnp.float32)
                + bf1_r[...], 0.0)
            lf_r[...] = jnp.dot(
                tf, wf2_r[...], preferred_element_type=jnp.float32) + bf2_r[...]
            tb = jnp.maximum(
                jnp.dot(h, wb1_r[...], preferred_element_type=jnp.float32)
                + bb1_r[...], 0.0)
            lb_r[...] = jnp.dot(
                tb, wb2_r[...], preferred_element_type=jnp.float32) + bb2_r[...]

    full = lambda shape: pl.BlockSpec(shape, lambda i: tuple(0 for _ in shape))
    return pl.pallas_call(
        body,
        grid=(NB,),
        in_specs=[
            pl.BlockSpec((1, 1, R), lambda i: (i, 0, 0)),
            pl.BlockSpec((R, 128), lambda i: (i, 0)),
            pl.BlockSpec((R, 128), lambda i: (i, 0)),
            pl.BlockSpec((R, 128), lambda i: (i, 0)),
            pl.BlockSpec((R, 128), lambda i: (i, 0)),
            full((1, DH)),
            full((2 * DH, DH)), full((1, DH)),
            full((DH, NT)), full((1, NT)),
            full((2 * DH, DH)), full((1, DH)),
            full((DH, NT)), full((1, NT)),
        ],
        out_specs=[full((G, NT)), full((G, NT))],
        out_shape=[
            jax.ShapeDtypeStruct((G, NT), jnp.float32),
            jax.ShapeDtypeStruct((G, NT), jnp.float32),
        ],
        scratch_shapes=[pltpu.VMEM((G, 2 * DH), jnp.float32)],
    )(batch3, h1a, h1b, h2a, h2b, bsub,
      Wf1T, bf1, Wf2T, bf2, Wb1T, bb1, Wb2T, bb2)


def kernel(x, y, edge_index, edge_attr, batch, emb, We1, be1, W1a, b1a, g1,
           bt1, W1b, b1b, We2, be2, W2a, b2a, g2, bt2, W2b, b2b, Wf1, bf1,
           Wf2, bf2, Wb1, bb1, Wb2, bb2):
    f32 = jnp.float32
    embp = jnp.pad(emb.astype(f32), ((0, 0), (0, 1)))
    xidx = x.reshape(-1).astype(jnp.int32)
    pad = EP - E
    src = jnp.concatenate([
        edge_index[0].astype(jnp.int32),
        jnp.arange(pad, dtype=jnp.int32) % N,
    ])
    dst = jnp.concatenate([
        edge_index[1].astype(jnp.int32),
        N + (jnp.arange(pad, dtype=jnp.int32) % 128),
    ])
    dst2 = dst.reshape(EP // CHE, CHE)
    ea = jnp.pad(edge_attr.reshape(-1).astype(f32), (0, pad))
    zeros = jnp.zeros((N, 128), f32)

    w1r = We1[:, 0].reshape(1, 128)
    b1r = be1.reshape(1, 128)
    w2h = We2[:, 0].reshape(2, 128)
    be2r = be2.reshape(1, -1)
    b1a_c = (b1a - be1 @ W1a.T).reshape(1, -1)
    b2a_c = (b2a - be2 @ W2a.T).reshape(1, -1)

    xy = _emb_gather(embp, xidx, y, b1r)
    pagg1 = _edge_pass1(xy, src, dst2, ea, w1r, zeros)

    u1, st1 = _mlp_a([xy], pagg1, W1a.T, b1a_c)
    h1a, h1b = _mlp_b(u1, st1, g1.reshape(1, -1), bt1.reshape(1, -1),
                      W1b.T, b1b.reshape(1, -1), be2r)

    agg2 = _edge_pass2(h1a, h1b, src, dst2, ea, w2h, zeros)
    u2, st2 = _mlp_a([h1a, h1b], agg2, W2a.T, b2a_c)
    h2a, h2b = _mlp_b(u2, st2, g2.reshape(1, -1), bt2.reshape(1, -1),
                      W2b.T, b2b.reshape(1, -1), jnp.zeros((1, DH), f32))

    batch3 = batch.reshape(NB, 1, R).astype(jnp.int32)
    lf, lb = _pool_heads(batch3, h1a, h1b, h2a, h2b, be2r,
                         Wf1.T, bf1.reshape(1, -1), Wf2.T, bf2.reshape(1, -1),
                         Wb1.T, bb1.reshape(1, -1), Wb2.T, bb2.reshape(1, -1))
    return (lf, lb)


# speedup vs baseline: 2.0282x; 1.0582x over previous
import functools

import jax
import jax.numpy as jnp
from jax import lax
from jax.experimental import pallas as pl
from jax.experimental.pallas import tpu as pltpu
from jax.experimental.pallas import tpu_sc as plsc

N = 10000
E = 160000
G = 128
NT = 512
DH = 256

NC = 2
NS = 16
NW = NC * NS

EP = 163840

_mesh = functools.partial(
    plsc.VectorSubcoreMesh,
    core_axis_name="c",
    subcore_axis_name="s",
    num_cores=NC,
    num_subcores=NS,
)

SUBQ = 632


def _splat16(val):
    return jnp.zeros((16,), jnp.int32) + val


def _lane(vec16, j):
    return lax.gather(
        vec16,
        _splat16(j)[:, None],
        lax.GatherDimensionNumbers(
            offset_dims=(), collapsed_slice_dims=(0,), start_index_map=(0,)),
        slice_sizes=(1,),
        mode=lax.GatherScatterMode.PROMISE_IN_BOUNDS,
    )


def _emb_gather(embp, xidx, y, b1r):
    QUOTA = 320
    CH = 80

    @functools.partial(
        pl.kernel,
        out_type=jax.ShapeDtypeStruct((N, 128), jnp.float32),
        mesh=_mesh(),
        scratch_types=[
            pltpu.VMEM((CH,), jnp.int32),
            pltpu.VMEM((CH,), jnp.float32),
            pltpu.VMEM((CH, 128), jnp.float32),
            pltpu.VMEM((1, 128), jnp.float32),
            pltpu.SemaphoreType.DMA,
        ],
    )
    def k(emb_h, idx_h, y_h, b_h, out_h, idxv, yv, rows, bvm, sem):
        cid = lax.axis_index("c")
        sid = lax.axis_index("s")
        wid = sid * NC + cid
        base = jnp.minimum(wid * QUOTA, N - QUOTA)
        pltpu.sync_copy(b_h, bvm)
        bb = [bvm[0, pl.ds(16 * c2, 16)] for c2 in range(8)]
        lastlane = lax.iota(jnp.int32, 16) == 15
        for kk in range(QUOTA // CH):
            b2 = base + kk * CH
            pltpu.sync_copy(idx_h.at[pl.ds(b2, CH)], idxv)
            pltpu.sync_copy(y_h.at[pl.ds(b2, CH)], yv)
            pltpu.async_copy(emb_h.at[idxv], rows, sem).wait()

            def grp(g, carry):
                y16 = yv[pl.ds(g * 16, 16)]
                for j in range(16):
                    r = g * 16 + j
                    yj = _lane(y16, j)
                    for c2 in range(8):
                        sl = pl.ds(16 * c2, 16)
                        v = rows[r, sl]
                        if c2 == 7:
                            v = jnp.where(lastlane, yj, v)
                        rows[r, sl] = v + bb[c2]
                return carry

            lax.fori_loop(0, CH // 16, grp, 0)
            pltpu.sync_copy(rows, out_h.at[pl.ds(b2, CH)])

    return k(embp, xidx, y, b1r)


CHE = 64
SECK = 40


def _edge_section(feat_h, aggr, srcv, eav, dstv, bufs, gsems, ssems, wv):
    def prep(k, b):
        pltpu.async_copy(
            feat_h.at[srcv.at[pl.ds(CHE * k, CHE)]], bufs[b], gsems[b])

    def waitg(b):
        pltpu.make_async_copy(
            feat_h.at[srcv.at[pl.ds(0, CHE)]], bufs[b], gsems[b]).wait()

    def scat(k, b):
        pltpu.async_copy(bufs[b], aggr.at[dstv.at[k]], ssems[b], add=True)

    def waits(b):
        pltpu.make_async_copy(bufs[b], aggr.at[dstv.at[0]], ssems[b]).wait()

    def compute(k, b):
        rg = bufs[b]

        def grp(g, carry):
            a16 = eav[pl.ds(CHE * k + g * 16, 16)]
            for j in range(16):
                e = g * 16 + j
                aj = _lane(a16, j)
                for c2 in range(8):
                    sl = pl.ds(16 * c2, 16)
                    rg[e, sl] = jnp.maximum(rg[e, sl] + aj * wv[c2], 0.0)
            return carry

        lax.fori_loop(0, CHE // 16, grp, 0)

    prep(0, 0)
    prep(1, 1)

    def lp(kk, carry):
        for b in range(4):
            k = 4 * kk + b
            bp = (b + 2) % 4

            @pl.when(k >= 2)
            def _():
                waits(bp)

            @pl.when(k + 2 < SECK)
            def _():
                prep(k + 2, bp)

            waitg(b)
            compute(k, b)
            scat(k, b)
        return carry

    lax.fori_loop(0, SECK // 4, lp, 0)
    waits(2)
    waits(3)


def _edge_scratch():
    return (
        [
            pltpu.VMEM_SHARED((N + 128, 128), jnp.float32),
            pltpu.VMEM((SECK * CHE,), jnp.int32),
            pltpu.VMEM((SECK * CHE,), jnp.float32),
            pltpu.VMEM((SECK, CHE), jnp.int32),
        ]
        + [pltpu.VMEM((CHE, 128), jnp.float32) for _ in range(4)]
        + [pltpu.VMEM((1, 128), jnp.float32)]
        + [pltpu.SemaphoreType.DMA] * 8
    )


def _zero_aggr(aggr, b0, sid, r0):
    def zr(r, carry):
        for c2 in range(8):
            b0[r, pl.ds(16 * c2, 16)] = jnp.zeros((16,), jnp.float32)
        return carry

    lax.fori_loop(0, 64, zr, 0)
    for i in range(10):
        off = min(64 * i, SUBQ - 64)
        pltpu.sync_copy(b0, aggr.at[pl.ds(r0 + off, 64)])

    @pl.when(sid == 0)
    def _():
        pltpu.sync_copy(b0, aggr.at[pl.ds(N, 64)])
        pltpu.sync_copy(b0, aggr.at[pl.ds(N + 64, 64)])


def _edge_pass1(xy, src, dst2, ea, w1r):
    @functools.partial(
        pl.kernel,
        out_type=jax.ShapeDtypeStruct((2, N, 128), jnp.float32),
        mesh=_mesh(),
        scratch_types=_edge_scratch(),
    )
    def k(xy_h, src_h, dst_h, ea_h, w_h, out_h,
          aggr, srcv, eav, dstv, b0, b1, b2, b3, wvm,
          g0, g1, g2, g3, s0, s1, s2, s3):
        cid = lax.axis_index("c")
        sid = lax.axis_index("s")
        wid = sid * NC + cid
        r0 = jnp.minimum(sid * SUBQ, N - SUBQ)
        _zero_aggr(aggr, b0, sid, r0)
        pltpu.sync_copy(w_h, wvm)
        plsc.subcore_barrier()
        wv = [wvm[0, pl.ds(16 * c2, 16)] for c2 in range(8)]

        def section(s, carry):
            eb = wid * (2 * SECK * CHE) + s * (SECK * CHE)
            pltpu.sync_copy(src_h.at[pl.ds(eb, SECK * CHE)], srcv)
            pltpu.sync_copy(ea_h.at[pl.ds(eb, SECK * CHE)], eav)
            pltpu.sync_copy(
                dst_h.at[pl.ds(wid * (2 * SECK) + s * SECK, SECK)], dstv)
            _edge_section(xy_h, aggr, srcv, eav, dstv,
                          (b0, b1, b2, b3), (g0, g1, g2, g3),
                          (s0, s1, s2, s3), wv)
            return carry

        lax.fori_loop(0, 2, section, 0)
        plsc.subcore_barrier()
        pltpu.sync_copy(aggr.at[pl.ds(r0, SUBQ)],
                        out_h.at[cid, pl.ds(r0, SUBQ)])

    return k(xy, src, dst2, ea, w1r)


def _edge_pass2(h1a, h1b, src, dst2, ea, w2h):
    @functools.partial(
        pl.kernel,
        out_type=jax.ShapeDtypeStruct((2, N, 128), jnp.float32),
        mesh=_mesh(),
        scratch_types=_edge_scratch(),
    )
    def k(ha_h, hb_h, src_h, dst_h, ea_h, w_h, out_h,
          aggr, srcv, eav, dstv, b0, b1, b2, b3, wvm,
          g0, g1, g2, g3, s0, s1, s2, s3):
        cid = lax.axis_index("c")
        sid = lax.axis_index("s")
        r0 = jnp.minimum(sid * SUBQ, N - SUBQ)
        _zero_aggr(aggr, b0, sid, r0)
        pltpu.sync_copy(w_h.at[pl.ds(cid, 1)], wvm)
        plsc.subcore_barrier()
        wv = [wvm[0, pl.ds(16 * c2, 16)] for c2 in range(8)]

        def section(s, carry):
            eb = sid * (4 * SECK * CHE) + s * (SECK * CHE)
            pltpu.sync_copy(src_h.at[pl.ds(eb, SECK * CHE)], srcv)
            pltpu.sync_copy(ea_h.at[pl.ds(eb, SECK * CHE)], eav)
            pltpu.sync_copy(
                dst_h.at[pl.ds(sid * (4 * SECK) + s * SECK, SECK)], dstv)

            @pl.when(cid == 0)
            def _():
                _edge_section(ha_h, aggr, srcv, eav, dstv,
                              (b0, b1, b2, b3), (g0, g1, g2, g3),
                              (s0, s1, s2, s3), wv)

            @pl.when(cid == 1)
            def _():
                _edge_section(hb_h, aggr, srcv, eav, dstv,
                              (b0, b1, b2, b3), (g0, g1, g2, g3),
                              (s0, s1, s2, s3), wv)

            return carry

        lax.fori_loop(0, 4, section, 0)
        plsc.subcore_barrier()
        pltpu.sync_copy(aggr.at[pl.ds(r0, SUBQ)],
                        out_h.at[cid, pl.ds(r0, SUBQ)])

    return k(h1a, h1b, src, dst2, ea, w2h)


R = 1000
NB = N // R


def _bn_scale(acc_r, g_r, bt_r):
    st = acc_r[...]
    m = st[0:1, :] / N
    v = st[1:2, :] / N - m * m
    inv = lax.rsqrt(v + 1e-5)
    return m, inv * g_r[...], bt_r[...]


def _stats_step(u, acc_r, j):
    s1 = jnp.sum(u, axis=0, keepdims=True)
    s2 = jnp.sum(u * u, axis=0, keepdims=True)
    st = jnp.concatenate([s1, s2], axis=0)

    @pl.when(j == 0)
    def _():
        acc_r[...] = st

    @pl.when(j > 0)
    def _():
        acc_r[...] = acc_r[...] + st


def _full(shape):
    return pl.BlockSpec(shape, lambda p, j: tuple(0 for _ in shape))


def _layer1(xy, pagg, be1r, W1aT, b1a, g1, bt1, W1bT, b1b, be2r):
    def body(xy_r, pa_r, be1_r, wa_r, ba_r, g_r, bt_r, wb_r, bb_r, be2_r,
             ha_r, hb_r, u_scr, acc_r):
        p = pl.program_id(0)
        j = pl.program_id(1)

        @pl.when(p == 0)
        def _():
            z = xy_r[...] + pa_r[0] + pa_r[1] - be1_r[...]
            u = jnp.dot(z, wa_r[...],
                        preferred_element_type=jnp.float32) + ba_r[...]
            u_scr[pl.ds(j * R, R), :] = u
            _stats_step(u, acc_r, j)

        @pl.when(p == 1)
        def _():
            m, sc, sh = _bn_scale(acc_r, g_r, bt_r)
            u = u_scr[pl.ds(j * R, R), :]
            t = jnp.maximum((u - m) * sc + sh, 0.0)
            h = jnp.dot(t, wb_r[...],
                        preferred_element_type=jnp.float32) + bb_r[...]
            h = jnp.maximum(h, 0.0) + be2_r[...]
            ha_r[...] = h[:, :128]
            hb_r[...] = h[:, 128:]

    return pl.pallas_call(
        body,
        grid=(2, NB),
        in_specs=[
            pl.BlockSpec((R, 128), lambda p, j: (j, 0)),
            pl.BlockSpec((2, R, 128), lambda p, j: (0, j, 0)),
            _full((1, 128)),
            _full((128, DH)), _full((1, DH)),
            _full((1, DH)), _full((1, DH)),
            _full((DH, DH)), _full((1, DH)), _full((1, DH)),
        ],
        out_specs=[
            pl.BlockSpec((R, 128), lambda p, j: (j, 0)),
            pl.BlockSpec((R, 128), lambda p, j: (j, 0)),
        ],
        out_shape=[
            jax.ShapeDtypeStruct((N, 128), jnp.float32),
            jax.ShapeDtypeStruct((N, 128), jnp.float32),
        ],
        scratch_shapes=[
            pltpu.VMEM((N, DH), jnp.float32),
            pltpu.VMEM((2, DH), jnp.float32),
        ],
    )(xy, pagg, be1r, W1aT, b1a, g1, bt1, W1bT, b1b, be2r)


def _layer2_heads(h1a, h1b, agg2, be2r, batch3, W2aT, b2a, g2, bt2, W2bT, b2b,
                  Wf1T, bf1, Wf2T, bf2, Wb1T, bb1, Wb2T, bb2):
    def body(ha_r, hb_r, ag_r, be2_r, b_r, wa_r, ba_r, g_r, bt_r, wb_r, bb_r,
             wf1_r, bf1_r, wf2_r, bf2_r, wb1_r, bb1_r, wb2_r, bb2_r,
             lf_r, lb_r, u_scr, acc_r, pacc):
        p = pl.program_id(0)
        j = pl.program_id(1)
        oh = (lax.broadcasted_iota(jnp.int32, (G, R), 0) == b_r[0])
        oh = oh.astype(jnp.float32)

        def pool(col, blk):
            sl = pl.ds(128 * col, 128)
            pacc[:, sl] = pacc[:, sl] + jnp.dot(
                oh, blk, preferred_element_type=jnp.float32)

        @pl.when(p == 0)
        def _():
            @pl.when(j == 0)
            def _():
                pacc[...] = jnp.zeros((G, 4 * 128), jnp.float32)

            ha = ha_r[...]
            hb = hb_r[...]
            z = jnp.concatenate([ha, hb], axis=1) \
                + jnp.concatenate([ag_r[0], ag_r[1]], axis=1) - be2_r[...]
            u = jnp.dot(z, wa_r[...],
                        preferred_element_type=jnp.float32) + ba_r[...]
            u_scr[pl.ds(j * R, R), :] = u
            _stats_step(u, acc_r, j)
            pool(0, ha - be2_r[:, :128])
            pool(1, hb - be2_r[:, 128:])

        @pl.when(p == 1)
        def _():
            m, sc, sh = _bn_scale(acc_r, g_r, bt_r)
            u = u_scr[pl.ds(j * R, R), :]
            t = jnp.maximum((u - m) * sc + sh, 0.0)
            h2 = jnp.dot(t, wb_r[...],
                         preferred_element_type=jnp.float32) + bb_r[...]
            h2 = jnp.maximum(h2, 0.0)
            pool(2, h2[:, :128])
            pool(3, h2[:, 128:])

            @pl.when(j == NB - 1)
            def _():
                hp = pacc[...]
                tf = jnp.maximum(
                    jnp.dot(hp, wf1_r[...],
                            preferred_element_type=jnp.float32)
                    + bf1_r[...], 0.0)
                lf_r[...] = jnp.dot(
                    tf, wf2_r[...],
                    preferred_element_type=jnp.float32) + bf2_r[...]
                tb = jnp.maximum(
                    jnp.dot(hp, wb1_r[...],
                            preferred_element_type=jnp.float32)
                    + bb1_r[...], 0.0)
                lb_r[...] = jnp.dot(
                    tb, wb2_r[...],
                    preferred_element_type=jnp.float32) + bb2_r[...]

    return pl.pallas_call(
        body,
        grid=(2, NB),
        in_specs=[
            pl.BlockSpec((R, 128), lambda p, j: (j, 0)),
            pl.BlockSpec((R, 128), lambda p, j: (j, 0)),
            pl.BlockSpec((2, R, 128), lambda p, j: (0, j, 0)),
            _full((1, DH)),
            pl.BlockSpec((1, 1, R), lambda p, j: (j, 0, 0)),
            _full((DH, DH)), _full((1, DH)),
            _full((1, DH)), _full((1, DH)),
            _full((DH, DH)), _full((1, DH)),
            _full((2 * DH, DH)), _full((1, DH)),
            _full((DH, NT)), _full((1, NT)),
            _full((2 * DH, DH)), _full((1, DH)),
            _full((DH, NT)), _full((1, NT)),
        ],
        out_specs=[_full((G, NT)), _full((G, NT))],
        out_shape=[
            jax.ShapeDtypeStruct((G, NT), jnp.float32),
            jax.ShapeDtypeStruct((G, NT), jnp.float32),
        ],
        scratch_shapes=[
            pltpu.VMEM((N, DH), jnp.float32),
            pltpu.VMEM((2, DH), jnp.float32),
            pltpu.VMEM((G, 4 * 128), jnp.float32),
        ],
    )(h1a, h1b, agg2, be2r, batch3, W2aT, b2a, g2, bt2, W2bT, b2b,
      Wf1T, bf1, Wf2T, bf2, Wb1T, bb1, Wb2T, bb2)


def kernel(x, y, edge_index, edge_attr, batch, emb, We1, be1, W1a, b1a, g1,
           bt1, W1b, b1b, We2, be2, W2a, b2a, g2, bt2, W2b, b2b, Wf1, bf1,
           Wf2, bf2, Wb1, bb1, Wb2, bb2):
    f32 = jnp.float32
    embp = jnp.pad(emb.astype(f32), ((0, 0), (0, 1)))
    xidx = x.reshape(-1).astype(jnp.int32)
    pad = EP - E
    src = jnp.concatenate([
        edge_index[0].astype(jnp.int32),
        jnp.arange(pad, dtype=jnp.int32) % N,
    ])
    dst = jnp.concatenate([
        edge_index[1].astype(jnp.int32),
        N + (jnp.arange(pad, dtype=jnp.int32) % 128),
    ])
    dst2 = dst.reshape(EP // CHE, CHE)
    ea = jnp.pad(edge_attr.reshape(-1).astype(f32), (0, pad))

    w1r = We1[:, 0].reshape(1, 128)
    b1r = be1.reshape(1, 128)
    w2h = We2[:, 0].reshape(2, 128)
    be2r = be2.reshape(1, -1)

    xy = _emb_gather(embp, xidx, y, b1r)
    pagg1 = _edge_pass1(xy, src, dst2, ea, w1r)
    h1a, h1b = _layer1(xy, pagg1, b1r, W1a.T, b1a.reshape(1, -1),
                       g1.reshape(1, -1), bt1.reshape(1, -1),
                       W1b.T, b1b.reshape(1, -1), be2r)

    agg2 = _edge_pass2(h1a, h1b, src, dst2, ea, w2h)
    batch3 = batch.reshape(NB, 1, R).astype(jnp.int32)
    lf, lb = _layer2_heads(
        h1a, h1b, agg2, be2r, batch3, W2a.T, b2a.reshape(1, -1),
        g2.reshape(1, -1), bt2.reshape(1, -1), W2b.T, b2b.reshape(1, -1),
        Wf1.T, bf1.reshape(1, -1), Wf2.T, bf2.reshape(1, -1),
        Wb1.T, bb1.reshape(1, -1), Wb2.T, bb2.reshape(1, -1))
    return (lf, lb)


# speedup vs baseline: 2.0347x; 1.0032x over previous
import functools

import jax
import jax.numpy as jnp
from jax import lax
from jax.experimental import pallas as pl
from jax.experimental.pallas import tpu as pltpu
from jax.experimental.pallas import tpu_sc as plsc

N = 10000
E = 160000
G = 128
NT = 512
DH = 256

NC = 2
NS = 16
NW = NC * NS

EP = 163840

_mesh = functools.partial(
    plsc.VectorSubcoreMesh,
    core_axis_name="c",
    subcore_axis_name="s",
    num_cores=NC,
    num_subcores=NS,
)

SUBQ = 632


def _splat16(val):
    return jnp.zeros((16,), jnp.int32) + val


def _lane(vec16, j):
    return lax.gather(
        vec16,
        _splat16(j)[:, None],
        lax.GatherDimensionNumbers(
            offset_dims=(), collapsed_slice_dims=(0,), start_index_map=(0,)),
        slice_sizes=(1,),
        mode=lax.GatherScatterMode.PROMISE_IN_BOUNDS,
    )


def _emb_gather(embp, xidx, y, b1r):
    QUOTA = 320
    CH = 80

    @functools.partial(
        pl.kernel,
        out_type=jax.ShapeDtypeStruct((N, 128), jnp.float32),
        mesh=_mesh(),
        scratch_types=[
            pltpu.VMEM((CH,), jnp.int32),
            pltpu.VMEM((CH,), jnp.float32),
            pltpu.VMEM((CH, 128), jnp.float32),
            pltpu.VMEM((1, 128), jnp.float32),
            pltpu.SemaphoreType.DMA,
        ],
    )
    def k(emb_h, idx_h, y_h, b_h, out_h, idxv, yv, rows, bvm, sem):
        cid = lax.axis_index("c")
        sid = lax.axis_index("s")
        wid = sid * NC + cid
        base = jnp.minimum(wid * QUOTA, N - QUOTA)
        pltpu.sync_copy(b_h, bvm)
        bb = [bvm[0, pl.ds(16 * c2, 16)] for c2 in range(8)]
        lastlane = lax.iota(jnp.int32, 16) == 15
        for kk in range(QUOTA // CH):
            b2 = base + kk * CH
            pltpu.sync_copy(idx_h.at[pl.ds(b2, CH)], idxv)
            pltpu.sync_copy(y_h.at[pl.ds(b2, CH)], yv)
            pltpu.async_copy(emb_h.at[idxv], rows, sem).wait()

            def grp(g, carry):
                y16 = yv[pl.ds(g * 16, 16)]
                for j in range(16):
                    r = g * 16 + j
                    yj = _lane(y16, j)
                    for c2 in range(8):
                        sl = pl.ds(16 * c2, 16)
                        v = rows[r, sl]
                        if c2 == 7:
                            v = jnp.where(lastlane, yj, v)
                        rows[r, sl] = v + bb[c2]
                return carry

            lax.fori_loop(0, CH // 16, grp, 0)
            pltpu.sync_copy(rows, out_h.at[pl.ds(b2, CH)])

    return k(embp, xidx, y, b1r)


CHE = 64
SECK = 40


def _edge_section(feat_h, aggr, srcv, eav, dstv, bufs, gsems, ssems, wv):
    def prep(k, b):
        pltpu.async_copy(
            feat_h.at[srcv.at[pl.ds(CHE * k, CHE)]], bufs[b], gsems[b])

    def waitg(b):
        pltpu.make_async_copy(
            feat_h.at[srcv.at[pl.ds(0, CHE)]], bufs[b], gsems[b]).wait()

    def scat(k, b):
        pltpu.async_copy(bufs[b], aggr.at[dstv.at[k]], ssems[b], add=True)

    def waits(b):
        pltpu.make_async_copy(bufs[b], aggr.at[dstv.at[0]], ssems[b]).wait()

    def compute(k, b):
        rg = bufs[b]

        def grp(g, carry):
            a16 = eav[pl.ds(CHE * k + g * 16, 16)]
            for j in range(16):
                e = g * 16 + j
                aj = _lane(a16, j)
                for c2 in range(8):
                    sl = pl.ds(16 * c2, 16)
                    rg[e, sl] = jnp.maximum(rg[e, sl] + aj * wv[c2], 0.0)
            return carry

        lax.fori_loop(0, CHE // 16, grp, 0)

    prep(0, 0)
    prep(1, 1)

    def lp(kk, carry):
        for b in range(4):
            k = 4 * kk + b
            bp = (b + 2) % 4

            @pl.when(k >= 2)
            def _():
                waits(bp)

            @pl.when(k + 2 < SECK)
            def _():
                prep(k + 2, bp)

            waitg(b)
            compute(k, b)
            scat(k, b)
        return carry

    lax.fori_loop(0, SECK // 4, lp, 0)
    waits(2)
    waits(3)


def _edge_scratch():
    return (
        [
            pltpu.VMEM_SHARED((N + 128, 128), jnp.float32),
            pltpu.VMEM((SECK * CHE,), jnp.int32),
            pltpu.VMEM((SECK * CHE,), jnp.float32),
            pltpu.VMEM((SECK, CHE), jnp.int32),
        ]
        + [pltpu.VMEM((CHE, 128), jnp.float32) for _ in range(4)]
        + [pltpu.VMEM((1, 128), jnp.float32)]
        + [pltpu.SemaphoreType.DMA] * 8
    )


def _zero_aggr(aggr, b0, sid, r0):
    def zr(r, carry):
        for c2 in range(8):
            b0[r, pl.ds(16 * c2, 16)] = jnp.zeros((16,), jnp.float32)
        return carry

    lax.fori_loop(0, 64, zr, 0)
    for i in range(10):
        off = min(64 * i, SUBQ - 64)
        pltpu.sync_copy(b0, aggr.at[pl.ds(r0 + off, 64)])

    @pl.when(sid == 0)
    def _():
        pltpu.sync_copy(b0, aggr.at[pl.ds(N, 64)])
        pltpu.sync_copy(b0, aggr.at[pl.ds(N + 64, 64)])


def _edge_pass1(xy, src, dst2, ea, w1r):
    @functools.partial(
        pl.kernel,
        out_type=jax.ShapeDtypeStruct((2, N, 128), jnp.float32),
        mesh=_mesh(),
        scratch_types=_edge_scratch(),
    )
    def k(xy_h, src_h, dst_h, ea_h, w_h, out_h,
          aggr, srcv, eav, dstv, b0, b1, b2, b3, wvm,
          g0, g1, g2, g3, s0, s1, s2, s3):
        cid = lax.axis_index("c")
        sid = lax.axis_index("s")
        wid = sid * NC + cid
        r0 = jnp.minimum(sid * SUBQ, N - SUBQ)
        _zero_aggr(aggr, b0, sid, r0)
        pltpu.sync_copy(w_h, wvm)
        plsc.subcore_barrier()
        wv = [wvm[0, pl.ds(16 * c2, 16)] for c2 in range(8)]

        def section(s, carry):
            eb = wid * (2 * SECK * CHE) + s * (SECK * CHE)
            pltpu.sync_copy(src_h.at[pl.ds(eb, SECK * CHE)], srcv)
            pltpu.sync_copy(ea_h.at[pl.ds(eb, SECK * CHE)], eav)
            pltpu.sync_copy(
                dst_h.at[pl.ds(wid * (2 * SECK) + s * SECK, SECK)], dstv)
            _edge_section(xy_h, aggr, srcv, eav, dstv,
                          (b0, b1, b2, b3), (g0, g1, g2, g3),
                          (s0, s1, s2, s3), wv)
            return carry

        lax.fori_loop(0, 2, section, 0)
        plsc.subcore_barrier()
        pltpu.sync_copy(aggr.at[pl.ds(r0, SUBQ)],
                        out_h.at[cid, pl.ds(r0, SUBQ)])

    return k(xy, src, dst2, ea, w1r)


def _edge_pass2(h1a, h1b, src, dst2, ea, w2h):
    @functools.partial(
        pl.kernel,
        out_type=jax.ShapeDtypeStruct((2, N, 128), jnp.float32),
        mesh=_mesh(),
        scratch_types=_edge_scratch(),
    )
    def k(ha_h, hb_h, src_h, dst_h, ea_h, w_h, out_h,
          aggr, srcv, eav, dstv, b0, b1, b2, b3, wvm,
          g0, g1, g2, g3, s0, s1, s2, s3):
        cid = lax.axis_index("c")
        sid = lax.axis_index("s")
        r0 = jnp.minimum(sid * SUBQ, N - SUBQ)
        _zero_aggr(aggr, b0, sid, r0)
        pltpu.sync_copy(w_h.at[pl.ds(cid, 1)], wvm)
        plsc.subcore_barrier()
        wv = [wvm[0, pl.ds(16 * c2, 16)] for c2 in range(8)]

        def section(s, carry):
            eb = sid * (4 * SECK * CHE) + s * (SECK * CHE)
            pltpu.sync_copy(src_h.at[pl.ds(eb, SECK * CHE)], srcv)
            pltpu.sync_copy(ea_h.at[pl.ds(eb, SECK * CHE)], eav)
            pltpu.sync_copy(
                dst_h.at[pl.ds(sid * (4 * SECK) + s * SECK, SECK)], dstv)

            @pl.when(cid == 0)
            def _():
                _edge_section(ha_h, aggr, srcv, eav, dstv,
                              (b0, b1, b2, b3), (g0, g1, g2, g3),
                              (s0, s1, s2, s3), wv)

            @pl.when(cid == 1)
            def _():
                _edge_section(hb_h, aggr, srcv, eav, dstv,
                              (b0, b1, b2, b3), (g0, g1, g2, g3),
                              (s0, s1, s2, s3), wv)

            return carry

        lax.fori_loop(0, 4, section, 0)
        plsc.subcore_barrier()
        pltpu.sync_copy(aggr.at[pl.ds(r0, SUBQ)],
                        out_h.at[cid, pl.ds(r0, SUBQ)])

    return k(h1a, h1b, src, dst2, ea, w2h)


R = 1000
NB = N // R


def _bn_scale(acc_r, g_r, bt_r):
    st = acc_r[...]
    m = st[0:1, :] / N
    v = st[1:2, :] / N - m * m
    inv = lax.rsqrt(v + 1e-5)
    return m, inv * g_r[...], bt_r[...]


def _stats_step(u, acc_r, j):
    s1 = jnp.sum(u, axis=0, keepdims=True)
    s2 = jnp.sum(u * u, axis=0, keepdims=True)
    st = jnp.concatenate([s1, s2], axis=0)

    @pl.when(j == 0)
    def _():
        acc_r[...] = st

    @pl.when(j > 0)
    def _():
        acc_r[...] = acc_r[...] + st


def _full(shape):
    return pl.BlockSpec(shape, lambda p, j: tuple(0 for _ in shape))


def _layer1(xy, pagg, be1r, W1aT, b1a, g1, bt1, W1bT, b1b, be2r):
    def body(xy_r, pa_r, be1_r, wa_r, ba_r, g_r, bt_r, wb_r, bb_r, be2_r,
             ha_r, hb_r, u_scr, acc_r):
        p = pl.program_id(0)
        j = pl.program_id(1)

        @pl.when(p == 0)
        def _():
            z = xy_r[...] + pa_r[0] + pa_r[1] - be1_r[...]
            u = jnp.dot(z, wa_r[...],
                        preferred_element_type=jnp.float32) + ba_r[...]
            u_scr[pl.ds(j * R, R), :] = u
            _stats_step(u, acc_r, j)

        @pl.when(p == 1)
        def _():
            m, sc, sh = _bn_scale(acc_r, g_r, bt_r)
            u = u_scr[pl.ds(j * R, R), :]
            t = jnp.maximum((u - m) * sc + sh, 0.0)
            h = jnp.dot(t, wb_r[...],
                        preferred_element_type=jnp.float32) + bb_r[...]
            h = jnp.maximum(h, 0.0) + be2_r[...]
            ha_r[...] = h[:, :128]
            hb_r[...] = h[:, 128:]

    return pl.pallas_call(
        body,
        grid=(2, NB),
        in_specs=[
            pl.BlockSpec((R, 128), lambda p, j: (j, 0)),
            pl.BlockSpec((2, R, 128), lambda p, j: (0, j, 0)),
            _full((1, 128)),
            _full((128, DH)), _full((1, DH)),
            _full((1, DH)), _full((1, DH)),
            _full((DH, DH)), _full((1, DH)), _full((1, DH)),
        ],
        out_specs=[
            pl.BlockSpec((R, 128), lambda p, j: (p * j, 0)),
            pl.BlockSpec((R, 128), lambda p, j: (p * j, 0)),
        ],
        out_shape=[
            jax.ShapeDtypeStruct((N, 128), jnp.float32),
            jax.ShapeDtypeStruct((N, 128), jnp.float32),
        ],
        scratch_shapes=[
            pltpu.VMEM((N, DH), jnp.float32),
            pltpu.VMEM((2, DH), jnp.float32),
        ],
    )(xy, pagg, be1r, W1aT, b1a, g1, bt1, W1bT, b1b, be2r)


def _pool1(batch3, h1a, h1b, be2r):
    def body(b_r, ha_r, hb_r, be2_r, out_r, pacc):
        j = pl.program_id(0)

        @pl.when(j == 0)
        def _():
            pacc[...] = jnp.zeros((G, DH), jnp.float32)

        oh = (lax.broadcasted_iota(jnp.int32, (G, R), 0) == b_r[0])
        oh = oh.astype(jnp.float32)
        pacc[:, :128] = pacc[:, :128] + jnp.dot(
            oh, ha_r[...] - be2_r[:, :128], preferred_element_type=jnp.float32)
        pacc[:, 128:] = pacc[:, 128:] + jnp.dot(
            oh, hb_r[...] - be2_r[:, 128:], preferred_element_type=jnp.float32)

        @pl.when(j == NB - 1)
        def _():
            out_r[...] = pacc[...]

    return pl.pallas_call(
        body,
        grid=(NB,),
        in_specs=[
            pl.BlockSpec((1, 1, R), lambda j: (j, 0, 0)),
            pl.BlockSpec((R, 128), lambda j: (j, 0)),
            pl.BlockSpec((R, 128), lambda j: (j, 0)),
            pl.BlockSpec((1, DH), lambda j: (0, 0)),
        ],
        out_specs=[pl.BlockSpec((G, DH), lambda j: (0, 0))],
        out_shape=[jax.ShapeDtypeStruct((G, DH), jnp.float32)],
        scratch_shapes=[pltpu.VMEM((G, DH), jnp.float32)],
    )(batch3, h1a, h1b, be2r)[0]


def _layer2_heads(h1a, h1b, agg2, p1, be2r, batch3, W2aT, b2a, g2, bt2,
                  W2bT, b2b, Wf1T, bf1, Wf2T, bf2, Wb1T, bb1, Wb2T, bb2):
    def body(ha_r, hb_r, ag_r, p1_r, be2_r, b_r, wa_r, ba_r, g_r, bt_r,
             wb_r, bb_r, wf1_r, bf1_r, wf2_r, bf2_r, wb1_r, bb1_r,
             wb2_r, bb2_r, lf_r, lb_r, u_scr, acc_r, pacc):
        p = pl.program_id(0)
        j = pl.program_id(1)

        @pl.when(p == 0)
        def _():
            @pl.when(j == 0)
            def _():
                pacc[:, :DH] = p1_r[...]
                pacc[:, DH:] = jnp.zeros((G, DH), jnp.float32)

            ha = ha_r[...]
            hb = hb_r[...]
            z = jnp.concatenate([ha, hb], axis=1) \
                + jnp.concatenate([ag_r[0], ag_r[1]], axis=1) - be2_r[...]
            u = jnp.dot(z, wa_r[...],
                        preferred_element_type=jnp.float32) + ba_r[...]
            u_scr[pl.ds(j * R, R), :] = u
            _stats_step(u, acc_r, j)

        @pl.when(p == 1)
        def _():
            oh = (lax.broadcasted_iota(jnp.int32, (G, R), 0) == b_r[0])
            oh = oh.astype(jnp.float32)

            def pool(col, blk):
                sl = pl.ds(128 * col, 128)
                pacc[:, sl] = pacc[:, sl] + jnp.dot(
                    oh, blk, preferred_element_type=jnp.float32)

            m, sc, sh = _bn_scale(acc_r, g_r, bt_r)
            u = u_scr[pl.ds(j * R, R), :]
            t = jnp.maximum((u - m) * sc + sh, 0.0)
            h2 = jnp.dot(t, wb_r[...],
                         preferred_element_type=jnp.float32) + bb_r[...]
            h2 = jnp.maximum(h2, 0.0)
            pool(2, h2[:, :128])
            pool(3, h2[:, 128:])

            @pl.when(j == NB - 1)
            def _():
                hp = pacc[...]
                tf = jnp.maximum(
                    jnp.dot(hp, wf1_r[...],
                            preferred_element_type=jnp.float32)
                    + bf1_r[...], 0.0)
                lf_r[...] = jnp.dot(
                    tf, wf2_r[...],
                    preferred_element_type=jnp.float32) + bf2_r[...]
                tb = jnp.maximum(
                    jnp.dot(hp, wb1_r[...],
                            preferred_element_type=jnp.float32)
                    + bb1_r[...], 0.0)
                lb_r[...] = jnp.dot(
                    tb, wb2_r[...],
                    preferred_element_type=jnp.float32) + bb2_r[...]

    return pl.pallas_call(
        body,
        grid=(2, NB),
        in_specs=[
            pl.BlockSpec((R, 128), lambda p, j: (j, 0)),
            pl.BlockSpec((R, 128), lambda p, j: (j, 0)),
            pl.BlockSpec((2, R, 128), lambda p, j: (0, j, 0)),
            _full((G, DH)),
            _full((1, DH)),
            pl.BlockSpec((1, 1, R), lambda p, j: (j, 0, 0)),
            _full((DH, DH)), _full((1, DH)),
            _full((1, DH)), _full((1, DH)),
            _full((DH, DH)), _full((1, DH)),
            _full((2 * DH, DH)), _full((1, DH)),
            _full((DH, NT)), _full((1, NT)),
            _full((2 * DH, DH)), _full((1, DH)),
            _full((DH, NT)), _full((1, NT)),
        ],
        out_specs=[_full((G, NT)), _full((G, NT))],
        out_shape=[
            jax.ShapeDtypeStruct((G, NT), jnp.float32),
            jax.ShapeDtypeStruct((G, NT), jnp.float32),
        ],
        scratch_shapes=[
            pltpu.VMEM((N, DH), jnp.float32),
            pltpu.VMEM((2, DH), jnp.float32),
            pltpu.VMEM((G, 4 * 128), jnp.float32),
        ],
    )(h1a, h1b, agg2, p1, be2r, batch3, W2aT, b2a, g2, bt2, W2bT, b2b,
      Wf1T, bf1, Wf2T, bf2, Wb1T, bb1, Wb2T, bb2)


def kernel(x, y, edge_index, edge_attr, batch, emb, We1, be1, W1a, b1a, g1,
           bt1, W1b, b1b, We2, be2, W2a, b2a, g2, bt2, W2b, b2b, Wf1, bf1,
           Wf2, bf2, Wb1, bb1, Wb2, bb2):
    f32 = jnp.float32
    embp = jnp.pad(emb.astype(f32), ((0, 0), (0, 1)))
    xidx = x.reshape(-1).astype(jnp.int32)
    pad = EP - E
    src = jnp.concatenate([
        edge_index[0].astype(jnp.int32),
        jnp.arange(pad, dtype=jnp.int32) % N,
    ])
    dst = jnp.concatenate([
        edge_index[1].astype(jnp.int32),
        N + (jnp.arange(pad, dtype=jnp.int32) % 128),
    ])
    dst2 = dst.reshape(EP // CHE, CHE)
    ea = jnp.pad(edge_attr.reshape(-1).astype(f32), (0, pad))

    w1r = We1[:, 0].reshape(1, 128)
    b1r = be1.reshape(1, 128)
    w2h = We2[:, 0].reshape(2, 128)
    be2r = be2.reshape(1, -1)

    xy = _emb_gather(embp, xidx, y, b1r)
    pagg1 = _edge_pass1(xy, src, dst2, ea, w1r)
    h1a, h1b = _layer1(xy, pagg1, b1r, W1a.T, b1a.reshape(1, -1),
                       g1.reshape(1, -1), bt1.reshape(1, -1),
                       W1b.T, b1b.reshape(1, -1), be2r)

    batch3 = batch.reshape(NB, 1, R).astype(jnp.int32)
    agg2 = _edge_pass2(h1a, h1b, src, dst2, ea, w2h)
    p1 = _pool1(batch3, h1a, h1b, be2r)
    lf, lb = _layer2_heads(
        h1a, h1b, agg2, p1, be2r, batch3, W2a.T, b2a.reshape(1, -1),
        g2.reshape(1, -1), bt2.reshape(1, -1), W2b.T, b2b.reshape(1, -1),
        Wf1.T, bf1.reshape(1, -1), Wf2.T, bf2.reshape(1, -1),
        Wb1.T, bb1.reshape(1, -1), Wb2.T, bb2.reshape(1, -1))
    return (lf, lb)


# speedup vs baseline: 2.1607x; 1.0620x over previous
import functools

import jax
import jax.numpy as jnp
from jax import lax
from jax.experimental import pallas as pl
from jax.experimental.pallas import tpu as pltpu
from jax.experimental.pallas import tpu_sc as plsc

N = 10000
E = 160000
G = 128
NT = 512
DH = 256

NC = 2
NS = 16
NW = NC * NS

EP = 163840

_mesh = functools.partial(
    plsc.VectorSubcoreMesh,
    core_axis_name="c",
    subcore_axis_name="s",
    num_cores=NC,
    num_subcores=NS,
)

SUBQ = 632


def _splat16(val):
    return jnp.zeros((16,), jnp.int32) + val


def _lane(vec16, j):
    return lax.gather(
        vec16,
        _splat16(j)[:, None],
        lax.GatherDimensionNumbers(
            offset_dims=(), collapsed_slice_dims=(0,), start_index_map=(0,)),
        slice_sizes=(1,),
        mode=lax.GatherScatterMode.PROMISE_IN_BOUNDS,
    )


def _emb_gather(embp, xidx, y, b1r):
    QUOTA = 320
    CH = 80

    @functools.partial(
        pl.kernel,
        out_type=jax.ShapeDtypeStruct((N, 128), jnp.float32),
        mesh=_mesh(),
        scratch_types=(
            [pltpu.VMEM((CH,), jnp.int32) for _ in range(4)]
            + [pltpu.VMEM((CH,), jnp.float32) for _ in range(4)]
            + [pltpu.VMEM((CH, 128), jnp.float32) for _ in range(4)]
            + [pltpu.VMEM((1, 128), jnp.float32)]
            + [pltpu.SemaphoreType.DMA for _ in range(4)]
        ),
    )
    def k(emb_h, idx_h, y_h, b_h, out_h,
          i0, i1, i2, i3, y0, y1, y2, y3, r0, r1, r2, r3, bvm,
          s0, s1, s2, s3):
        idxv = (i0, i1, i2, i3)
        yv = (y0, y1, y2, y3)
        rows = (r0, r1, r2, r3)
        sem = (s0, s1, s2, s3)
        cid = lax.axis_index("c")
        sid = lax.axis_index("s")
        wid = sid * NC + cid
        base = jnp.minimum(wid * QUOTA, N - QUOTA)
        pltpu.sync_copy(b_h, bvm)
        bb = [bvm[0, pl.ds(16 * c2, 16)] for c2 in range(8)]
        lastlane = lax.iota(jnp.int32, 16) == 15
        for kk in range(QUOTA // CH):
            b2 = base + kk * CH
            pltpu.sync_copy(idx_h.at[pl.ds(b2, CH)], idxv[kk])
            pltpu.sync_copy(y_h.at[pl.ds(b2, CH)], yv[kk])
            pltpu.async_copy(emb_h.at[idxv[kk]], rows[kk], sem[kk])
        for kk in range(QUOTA // CH):
            b2 = base + kk * CH
            pltpu.make_async_copy(
                emb_h.at[idxv[kk]], rows[kk], sem[kk]).wait()
            rr = rows[kk]
            yk = yv[kk]

            def grp(g, carry, rr=rr, yk=yk):
                y16 = yk[pl.ds(g * 16, 16)]
                for j in range(16):
                    r = g * 16 + j
                    yj = _lane(y16, j)
                    for c2 in range(8):
                        sl = pl.ds(16 * c2, 16)
                        v = rr[r, sl]
                        if c2 == 7:
                            v = jnp.where(lastlane, yj, v)
                        rr[r, sl] = v + bb[c2]
                return carry

            lax.fori_loop(0, CH // 16, grp, 0)
            pltpu.sync_copy(rr, out_h.at[pl.ds(b2, CH)])

    return k(embp, xidx, y, b1r)


CHE = 64
SECK = 40


def _edge_section(feat_h, aggr, srcv, eav, dstv, bufs, gsems, ssems, wv,
                  seck=SECK):
    def prep(k, b):
        pltpu.async_copy(
            feat_h.at[srcv.at[pl.ds(CHE * k, CHE)]], bufs[b], gsems[b])

    def waitg(b):
        pltpu.make_async_copy(
            feat_h.at[srcv.at[pl.ds(0, CHE)]], bufs[b], gsems[b]).wait()

    def scat(k, b):
        pltpu.async_copy(bufs[b], aggr.at[dstv.at[k]], ssems[b], add=True)

    def waits(b):
        pltpu.make_async_copy(bufs[b], aggr.at[dstv.at[0]], ssems[b]).wait()

    def compute(k, b):
        rg = bufs[b]

        def grp(g, carry):
            a16 = eav[pl.ds(CHE * k + g * 16, 16)]
            for j in range(16):
                e = g * 16 + j
                aj = _lane(a16, j)
                for c2 in range(8):
                    sl = pl.ds(16 * c2, 16)
                    rg[e, sl] = jnp.maximum(rg[e, sl] + aj * wv[c2], 0.0)
            return carry

        lax.fori_loop(0, CHE // 16, grp, 0)

    prep(0, 0)
    prep(1, 1)

    def lp(kk, carry):
        for b in range(4):
            k = 4 * kk + b
            bp = (b + 2) % 4

            @pl.when(k >= 2)
            def _():
                waits(bp)

            @pl.when(k + 2 < seck)
            def _():
                prep(k + 2, bp)

            waitg(b)
            compute(k, b)
            scat(k, b)
        return carry

    lax.fori_loop(0, seck // 4, lp, 0)
    waits(2)
    waits(3)


def _edge_scratch(seck=SECK):
    return (
        [
            pltpu.VMEM_SHARED((N + 128, 128), jnp.float32),
            pltpu.VMEM((seck * CHE,), jnp.int32),
            pltpu.VMEM((seck * CHE,), jnp.float32),
            pltpu.VMEM((seck, CHE), jnp.int32),
        ]
        + [pltpu.VMEM((CHE, 128), jnp.float32) for _ in range(4)]
        + [pltpu.VMEM((1, 128), jnp.float32)]
        + [pltpu.SemaphoreType.DMA] * 8
    )


def _zero_aggr(aggr, b0, sid, r0):
    def zr(r, carry):
        for c2 in range(8):
            b0[r, pl.ds(16 * c2, 16)] = jnp.zeros((16,), jnp.float32)
        return carry

    lax.fori_loop(0, 64, zr, 0)
    for i in range(10):
        off = min(64 * i, SUBQ - 64)
        pltpu.sync_copy(b0, aggr.at[pl.ds(r0 + off, 64)])

    @pl.when(sid == 0)
    def _():
        pltpu.sync_copy(b0, aggr.at[pl.ds(N, 64)])
        pltpu.sync_copy(b0, aggr.at[pl.ds(N + 64, 64)])


def _edge_pass1(xy, src, dst2, ea, w1r):
    @functools.partial(
        pl.kernel,
        out_type=jax.ShapeDtypeStruct((2, N, 128), jnp.float32),
        mesh=_mesh(),
        scratch_types=_edge_scratch(),
    )
    def k(xy_h, src_h, dst_h, ea_h, w_h, out_h,
          aggr, srcv, eav, dstv, b0, b1, b2, b3, wvm,
          g0, g1, g2, g3, s0, s1, s2, s3):
        cid = lax.axis_index("c")
        sid = lax.axis_index("s")
        wid = sid * NC + cid
        r0 = jnp.minimum(sid * SUBQ, N - SUBQ)
        _zero_aggr(aggr, b0, sid, r0)
        pltpu.sync_copy(w_h, wvm)
        plsc.subcore_barrier()
        wv = [wvm[0, pl.ds(16 * c2, 16)] for c2 in range(8)]

        def section(s, carry):
            eb = wid * (2 * SECK * CHE) + s * (SECK * CHE)
            pltpu.sync_copy(src_h.at[pl.ds(eb, SECK * CHE)], srcv)
            pltpu.sync_copy(ea_h.at[pl.ds(eb, SECK * CHE)], eav)
            pltpu.sync_copy(
                dst_h.at[pl.ds(wid * (2 * SECK) + s * SECK, SECK)], dstv)
            _edge_section(xy_h, aggr, srcv, eav, dstv,
                          (b0, b1, b2, b3), (g0, g1, g2, g3),
                          (s0, s1, s2, s3), wv)
            return carry

        lax.fori_loop(0, 2, section, 0)
        plsc.subcore_barrier()
        pltpu.sync_copy(aggr.at[pl.ds(r0, SUBQ)],
                        out_h.at[cid, pl.ds(r0, SUBQ)])

    return k(xy, src, dst2, ea, w1r)


def _edge_pass2(h1a, h1b, src, dst2, ea, w2h):
    @functools.partial(
        pl.kernel,
        out_type=jax.ShapeDtypeStruct((2, N, 128), jnp.float32),
        mesh=_mesh(),
        scratch_types=_edge_scratch(),
    )
    def k(ha_h, hb_h, src_h, dst_h, ea_h, w_h, out_h,
          aggr, srcv, eav, dstv, b0, b1, b2, b3, wvm,
          g0, g1, g2, g3, s0, s1, s2, s3):
        cid = lax.axis_index("c")
        sid = lax.axis_index("s")
        r0 = jnp.minimum(sid * SUBQ, N - SUBQ)
        _zero_aggr(aggr, b0, sid, r0)
        pltpu.sync_copy(w_h.at[pl.ds(cid, 1)], wvm)
        plsc.subcore_barrier()
        wv = [wvm[0, pl.ds(16 * c2, 16)] for c2 in range(8)]

        def section(s, carry):
            eb = sid * (4 * SECK * CHE) + s * (SECK * CHE)
            pltpu.sync_copy(src_h.at[pl.ds(eb, SECK * CHE)], srcv)
            pltpu.sync_copy(ea_h.at[pl.ds(eb, SECK * CHE)], eav)
            pltpu.sync_copy(
                dst_h.at[pl.ds(sid * (4 * SECK) + s * SECK, SECK)], dstv)

            @pl.when(cid == 0)
            def _():
                _edge_section(ha_h, aggr, srcv, eav, dstv,
                              (b0, b1, b2, b3), (g0, g1, g2, g3),
                              (s0, s1, s2, s3), wv)

            @pl.when(cid == 1)
            def _():
                _edge_section(hb_h, aggr, srcv, eav, dstv,
                              (b0, b1, b2, b3), (g0, g1, g2, g3),
                              (s0, s1, s2, s3), wv)

            return carry

        lax.fori_loop(0, 4, section, 0)
        plsc.subcore_barrier()
        pltpu.sync_copy(aggr.at[pl.ds(r0, SUBQ)],
                        out_h.at[cid, pl.ds(r0, SUBQ)])

    return k(h1a, h1b, src, dst2, ea, w2h)


R = 2000
NB = N // R


def _bn_scale(acc_r, g_r, bt_r):
    st = acc_r[...]
    m = st[0:1, :] / N
    v = st[1:2, :] / N - m * m
    inv = lax.rsqrt(v + 1e-5)
    return m, inv * g_r[...], bt_r[...]


def _stats_step(u, acc_r, j):
    s1 = jnp.sum(u, axis=0, keepdims=True)
    s2 = jnp.sum(u * u, axis=0, keepdims=True)
    st = jnp.concatenate([s1, s2], axis=0)

    @pl.when(j == 0)
    def _():
        acc_r[...] = st

    @pl.when(j > 0)
    def _():
        acc_r[...] = acc_r[...] + st


def _full(shape):
    return pl.BlockSpec(shape, lambda p, j: tuple(0 for _ in shape))


def _layer1(xy, pagg, be1r, W1aT, b1a, g1, bt1, W1bT, b1b, be2r):
    def body(xy_r, pa_r, be1_r, wa_r, ba_r, g_r, bt_r, wb_r, bb_r, be2_r,
             ha_r, hb_r, u_scr, acc_r):
        p = pl.program_id(0)
        j = pl.program_id(1)

        @pl.when(p == 0)
        def _():
            z = xy_r[...] + pa_r[0] + pa_r[1] - be1_r[...]
            u = jnp.dot(z, wa_r[...],
                        preferred_element_type=jnp.float32) + ba_r[...]
            u_scr[pl.ds(j * R, R), :] = u
            _stats_step(u, acc_r, j)

        @pl.when(p == 1)
        def _():
            m, sc, sh = _bn_scale(acc_r, g_r, bt_r)
            u = u_scr[pl.ds(j * R, R), :]
            t = jnp.maximum((u - m) * sc + sh, 0.0)
            h = jnp.dot(t, wb_r[...],
                        preferred_element_type=jnp.float32) + bb_r[...]
            h = jnp.maximum(h, 0.0) + be2_r[...]
            ha_r[...] = h[:, :128]
            hb_r[...] = h[:, 128:]

    return pl.pallas_call(
        body,
        grid=(2, NB),
        in_specs=[
            pl.BlockSpec((R, 128), lambda p, j: (j, 0)),
            pl.BlockSpec((2, R, 128), lambda p, j: (0, j, 0)),
            _full((1, 128)),
            _full((128, DH)), _full((1, DH)),
            _full((1, DH)), _full((1, DH)),
            _full((DH, DH)), _full((1, DH)), _full((1, DH)),
        ],
        out_specs=[
            pl.BlockSpec((R, 128), lambda p, j: (p * j, 0)),
            pl.BlockSpec((R, 128), lambda p, j: (p * j, 0)),
        ],
        out_shape=[
            jax.ShapeDtypeStruct((N, 128), jnp.float32),
            jax.ShapeDtypeStruct((N, 128), jnp.float32),
        ],
        scratch_shapes=[
            pltpu.VMEM((N, DH), jnp.float32),
            pltpu.VMEM((2, DH), jnp.float32),
        ],
    )(xy, pagg, be1r, W1aT, b1a, g1, bt1, W1bT, b1b, be2r)


def _pool1(batch3, h1a, h1b, be2r):
    def body(b_r, ha_r, hb_r, be2_r, out_r, pacc):
        j = pl.program_id(0)

        @pl.when(j == 0)
        def _():
            pacc[...] = jnp.zeros((G, DH), jnp.float32)

        oh = (lax.broadcasted_iota(jnp.int32, (G, R), 0) == b_r[0])
        oh = oh.astype(jnp.float32)
        pacc[:, :128] = pacc[:, :128] + jnp.dot(
            oh, ha_r[...] - be2_r[:, :128], preferred_element_type=jnp.float32)
        pacc[:, 128:] = pacc[:, 128:] + jnp.dot(
            oh, hb_r[...] - be2_r[:, 128:], preferred_element_type=jnp.float32)

        @pl.when(j == NB - 1)
        def _():
            out_r[...] = pacc[...]

    return pl.pallas_call(
        body,
        grid=(NB,),
        in_specs=[
            pl.BlockSpec((1, 1, R), lambda j: (j, 0, 0)),
            pl.BlockSpec((R, 128), lambda j: (j, 0)),
            pl.BlockSpec((R, 128), lambda j: (j, 0)),
            pl.BlockSpec((1, DH), lambda j: (0, 0)),
        ],
        out_specs=[pl.BlockSpec((G, DH), lambda j: (0, 0))],
        out_shape=[jax.ShapeDtypeStruct((G, DH), jnp.float32)],
        scratch_shapes=[pltpu.VMEM((G, DH), jnp.float32)],
    )(batch3, h1a, h1b, be2r)[0]


def _layer2_heads(h1a, h1b, agg2, p1, be2r, batch3, W2aT, b2a, g2, bt2,
                  W2bT, b2b, Wf1T, bf1, Wf2T, bf2, Wb1T, bb1, Wb2T, bb2):
    def body(ha_r, hb_r, ag_r, p1_r, be2_r, b_r, wa_r, ba_r, g_r, bt_r,
             wb_r, bb_r, wf1_r, bf1_r, wf2_r, bf2_r, wb1_r, bb1_r,
             wb2_r, bb2_r, lf_r, lb_r, u_scr, acc_r, pacc):
        p = pl.program_id(0)
        j = pl.program_id(1)

        @pl.when(p == 0)
        def _():
            @pl.when(j == 0)
            def _():
                pacc[:, :DH] = p1_r[...]
                pacc[:, DH:] = jnp.zeros((G, DH), jnp.float32)

            ha = ha_r[...]
            hb = hb_r[...]
            z = jnp.concatenate([ha, hb], axis=1) \
                + jnp.concatenate([ag_r[0], ag_r[1]], axis=1) - be2_r[...]
            u = jnp.dot(z, wa_r[...],
                        preferred_element_type=jnp.float32) + ba_r[...]
            u_scr[pl.ds(j * R, R), :] = u
            _stats_step(u, acc_r, j)

        @pl.when(p == 1)
        def _():
            oh = (lax.broadcasted_iota(jnp.int32, (G, R), 0) == b_r[0])
            oh = oh.astype(jnp.float32)

            def pool(col, blk):
                sl = pl.ds(128 * col, 128)
                pacc[:, sl] = pacc[:, sl] + jnp.dot(
                    oh, blk, preferred_element_type=jnp.float32)

            m, sc, sh = _bn_scale(acc_r, g_r, bt_r)
            u = u_scr[pl.ds(j * R, R), :]
            t = jnp.maximum((u - m) * sc + sh, 0.0)
            h2 = jnp.dot(t, wb_r[...],
                         preferred_element_type=jnp.float32) + bb_r[...]
            h2 = jnp.maximum(h2, 0.0)
            pool(2, h2[:, :128])
            pool(3, h2[:, 128:])

            @pl.when(j == NB - 1)
            def _():
                hp = pacc[...]
                tf = jnp.maximum(
                    jnp.dot(hp, wf1_r[...],
                            preferred_element_type=jnp.float32)
                    + bf1_r[...], 0.0)
                lf_r[...] = jnp.dot(
                    tf, wf2_r[...],
                    preferred_element_type=jnp.float32) + bf2_r[...]
                tb = jnp.maximum(
                    jnp.dot(hp, wb1_r[...],
                            preferred_element_type=jnp.float32)
                    + bb1_r[...], 0.0)
                lb_r[...] = jnp.dot(
                    tb, wb2_r[...],
                    preferred_element_type=jnp.float32) + bb2_r[...]

    return pl.pallas_call(
        body,
        grid=(2, NB),
        in_specs=[
            pl.BlockSpec((R, 128), lambda p, j: (j, 0)),
            pl.BlockSpec((R, 128), lambda p, j: (j, 0)),
            pl.BlockSpec((2, R, 128), lambda p, j: (0, j, 0)),
            _full((G, DH)),
            _full((1, DH)),
            pl.BlockSpec((1, 1, R), lambda p, j: (j, 0, 0)),
            _full((DH, DH)), _full((1, DH)),
            _full((1, DH)), _full((1, DH)),
            _full((DH, DH)), _full((1, DH)),
            _full((2 * DH, DH)), _full((1, DH)),
            _full((DH, NT)), _full((1, NT)),
            _full((2 * DH, DH)), _full((1, DH)),
            _full((DH, NT)), _full((1, NT)),
        ],
        out_specs=[_full((G, NT)), _full((G, NT))],
        out_shape=[
            jax.ShapeDtypeStruct((G, NT), jnp.float32),
            jax.ShapeDtypeStruct((G, NT), jnp.float32),
        ],
        scratch_shapes=[
            pltpu.VMEM((N, DH), jnp.float32),
            pltpu.VMEM((2, DH), jnp.float32),
            pltpu.VMEM((G, 4 * 128), jnp.float32),
        ],
    )(h1a, h1b, agg2, p1, be2r, batch3, W2aT, b2a, g2, bt2, W2bT, b2b,
      Wf1T, bf1, Wf2T, bf2, Wb1T, bb1, Wb2T, bb2)


def kernel(x, y, edge_index, edge_attr, batch, emb, We1, be1, W1a, b1a, g1,
           bt1, W1b, b1b, We2, be2, W2a, b2a, g2, bt2, W2b, b2b, Wf1, bf1,
           Wf2, bf2, Wb1, bb1, Wb2, bb2):
    f32 = jnp.float32
    embp = jnp.pad(emb.astype(f32), ((0, 0), (0, 1)))
    xidx = x.reshape(-1).astype(jnp.int32)
    pad = EP - E
    src = jnp.concatenate([
        edge_index[0].astype(jnp.int32),
        jnp.arange(pad, dtype=jnp.int32) % N,
    ])
    dst = jnp.concatenate([
        edge_index[1].astype(jnp.int32),
        N + (jnp.arange(pad, dtype=jnp.int32) % 128),
    ])
    dst2 = dst.reshape(EP // CHE, CHE)
    ea = jnp.pad(edge_attr.reshape(-1).astype(f32), (0, pad))

    w1r = We1[:, 0].reshape(1, 128)
    b1r = be1.reshape(1, 128)
    w2h = We2[:, 0].reshape(2, 128)
    be2r = be2.reshape(1, -1)

    xy = _emb_gather(embp, xidx, y, b1r)
    pagg1 = _edge_pass1(xy, src, dst2, ea, w1r)
    h1a, h1b = _layer1(xy, pagg1, b1r, W1a.T, b1a.reshape(1, -1),
                       g1.reshape(1, -1), bt1.reshape(1, -1),
                       W1b.T, b1b.reshape(1, -1), be2r)

    batch3 = batch.reshape(NB, 1, R).astype(jnp.int32)
    agg2 = _edge_pass2(h1a, h1b, src, dst2, ea, w2h)
    p1 = _pool1(batch3, h1a, h1b, be2r)
    lf, lb = _layer2_heads(
        h1a, h1b, agg2, p1, be2r, batch3, W2a.T, b2a.reshape(1, -1),
        g2.reshape(1, -1), bt2.reshape(1, -1), W2b.T, b2b.reshape(1, -1),
        Wf1.T, bf1.reshape(1, -1), Wf2.T, bf2.reshape(1, -1),
        Wb1.T, bb1.reshape(1, -1), Wb2.T, bb2.reshape(1, -1))
    return (lf, lb)
